# baseline (speedup 1.0000x reference)
import functools

import jax
import jax.numpy as jnp
from jax import lax
from jax.experimental import pallas as pl
from jax.experimental.pallas import tpu as pltpu
from jax.experimental.pallas import tpu_sc as plsc

VOCAB = 50257
D = 768
BATCH = 4
SEQ = 2048
TOT = BATCH * SEQ
NC = 2
NS = 16
NW = NC * NS
BPW = TOT // NW
C = 64
NCHUNK = BPW // C
LANES = 16
VECS_PER_ROW = D // LANES


def _emb_body(ids_hbm, wte_hbm, wpe_hbm, out_hbm, idx_v, rows_v, wpe_v, sem):
    wid = lax.axis_index("s") * NC + lax.axis_index("c")
    base = wid * BPW
    pos_base = base % SEQ

    for ci in range(NCHUNK):
        cbase = ci * C
        pltpu.sync_copy(ids_hbm.at[pl.ds(base + cbase, C)], idx_v.at[ci])
        gather = pltpu.async_copy(wte_hbm.at[idx_v.at[ci]], rows_v, sem)
        pltpu.sync_copy(wpe_hbm.at[pl.ds(pos_base + cbase, C)], wpe_v)
        gather.wait()

        def add_row(r, carry):
            def add_vec(j, c2):
                off = j * LANES
                rows_v[r, pl.ds(off, LANES)] = (
                    rows_v[r, pl.ds(off, LANES)] + wpe_v[r, pl.ds(off, LANES)]
                )
                return c2

            return lax.fori_loop(0, VECS_PER_ROW, add_vec, carry)

        lax.fori_loop(0, C, add_row, 0)
        pltpu.sync_copy(rows_v, out_hbm.at[pl.ds(base + cbase, C)])


@jax.jit
def _emb(flat_ids, wte, wpe):
    mesh = plsc.VectorSubcoreMesh(
        core_axis_name="c", subcore_axis_name="s", num_cores=NC, num_subcores=NS
    )
    return pl.kernel(
        _emb_body,
        out_type=jax.ShapeDtypeStruct((TOT, D), jnp.float32),
        mesh=mesh,
        scratch_types=[
            pltpu.VMEM((NCHUNK, C), jnp.int32),
            pltpu.VMEM((C, D), jnp.float32),
            pltpu.VMEM((C, D), jnp.float32),
            pltpu.SemaphoreType.DMA,
        ],
    )(flat_ids, wte, wpe)


def kernel(input_ids, wte, wpe):
    flat_ids = input_ids.reshape(TOT)
    out = _emb(flat_ids, wte, wpe)
    return out.reshape(BATCH, SEQ, D)


# speedup vs baseline: 1.8528x; 1.8528x over previous
import jax
import jax.numpy as jnp
from jax import lax
from jax.experimental import pallas as pl
from jax.experimental.pallas import tpu as pltpu
from jax.experimental.pallas import tpu_sc as plsc

VOCAB = 50257
D = 768
BATCH = 4
SEQ = 2048
TOT = BATCH * SEQ
NC = 2
NS = 16
NW = NC * NS
BPW = TOT // NW
C = 32
NCHUNK = BPW // C
LANES = 16
VECS_PER_ROW = D // LANES


def _emb_body(ids_hbm, wte_hbm, wpe_hbm, out_hbm,
              idx_v, r0, r1, w0, w1, w2,
              g0, g1, ws0, ws1, ws2, ss0, ss1, ss2):
    rows = [r0, r1]
    wpes = [w0, w1, w2]
    gsems = [g0, g1]
    wsems = [ws0, ws1, ws2]
    ssems = [ss0, ss1, ss2]

    wid = lax.axis_index("s") * NC + lax.axis_index("c")
    base = wid * BPW
    pos_base = base % SEQ

    pltpu.sync_copy(ids_hbm.at[pl.ds(base, BPW)], idx_v)

    def start_fetch(ci):
        g = pltpu.async_copy(
            wte_hbm.at[idx_v.at[pl.ds(ci * C, C)]], rows[ci % 2], gsems[ci % 2])
        w = pltpu.async_copy(
            wpe_hbm.at[pl.ds(pos_base + ci * C, C)], wpes[ci % 3], wsems[ci % 3])
        return g, w

    gdesc = [None, None]
    wdesc = [None, None, None]
    sdesc = [None, None, None]
    gdesc[0], wdesc[0] = start_fetch(0)

    for ci in range(NCHUNK):
        cur2 = ci % 2
        cur3 = ci % 3
        if ci + 1 < NCHUNK:
            nb3 = (ci + 1) % 3
            if sdesc[nb3] is not None:
                sdesc[nb3].wait()
            gdesc[(ci + 1) % 2], wdesc[nb3] = start_fetch(ci + 1)
        wdesc[cur3].wait()
        gdesc[cur2].wait()

        rbuf = rows[cur2]
        wbuf = wpes[cur3]

        @plsc.parallel_loop(0, C, unroll=2)
        def add_row(r):
            for j in range(VECS_PER_ROW):
                off = j * LANES
                v = rbuf[r, pl.ds(off, LANES)]
                plsc.addupdate(wbuf.at[r, pl.ds(off, LANES)], v)

        sdesc[cur3] = pltpu.async_copy(
            wbuf, out_hbm.at[pl.ds(base + ci * C, C)], ssems[cur3])

    sdesc[(NCHUNK - 2) % 3].wait()
    sdesc[(NCHUNK - 1) % 3].wait()


@jax.jit
def _emb(flat_ids, wte, wpe):
    mesh = plsc.VectorSubcoreMesh(
        core_axis_name="c", subcore_axis_name="s", num_cores=NC, num_subcores=NS
    )
    return pl.kernel(
        _emb_body,
        out_type=jax.ShapeDtypeStruct((TOT, D), jnp.float32),
        mesh=mesh,
        scratch_types=[
            pltpu.VMEM((BPW,), jnp.int32),
            pltpu.VMEM((C, D), jnp.float32),
            pltpu.VMEM((C, D), jnp.float32),
            pltpu.VMEM((C, D), jnp.float32),
            pltpu.VMEM((C, D), jnp.float32),
            pltpu.VMEM((C, D), jnp.float32),
            pltpu.SemaphoreType.DMA,
            pltpu.SemaphoreType.DMA,
            pltpu.SemaphoreType.DMA,
            pltpu.SemaphoreType.DMA,
            pltpu.SemaphoreType.DMA,
            pltpu.SemaphoreType.DMA,
            pltpu.SemaphoreType.DMA,
            pltpu.SemaphoreType.DMA,
        ],
    )(flat_ids, wte, wpe)


def kernel(input_ids, wte, wpe):
    flat_ids = input_ids.reshape(TOT)
    out = _emb(flat_ids, wte, wpe)
    return out.reshape(BATCH, SEQ, D)


# speedup vs baseline: 2.0798x; 1.1225x over previous
import jax
import jax.numpy as jnp
from jax import lax
from jax.experimental import pallas as pl
from jax.experimental.pallas import tpu as pltpu
from jax.experimental.pallas import tpu_sc as plsc

VOCAB = 50257
D = 768
BATCH = 4
SEQ = 2048
TOT = BATCH * SEQ
NC = 2
NS = 16
NW = NC * NS
PPW = SEQ // NW
C = 32
HALVES = PPW // C
NCHUNK = BATCH * HALVES
LANES = 16
VECS_PER_ROW = D // LANES
NRBUF = 3


def _emb_body(ids_hbm, wte_hbm, wpe_hbm, out_hbm,
              idx_v, wpe_v, r0, r1, r2,
              g0, g1, g2, s0, s1, s2, wsem):
    rows = [r0, r1, r2]
    gsems = [g0, g1, g2]
    ssems = [s0, s1, s2]

    wid = lax.axis_index("s") * NC + lax.axis_index("c")
    pos_base = wid * PPW

    wdesc = pltpu.async_copy(wpe_hbm.at[pl.ds(pos_base, PPW)], wpe_v, wsem)
    for b in range(BATCH):
        pltpu.sync_copy(ids_hbm.at[pl.ds(b * SEQ + pos_base, PPW)],
                        idx_v.at[pl.ds(b * PPW, PPW)])

    def start_gather(ci):
        return pltpu.async_copy(
            wte_hbm.at[idx_v.at[pl.ds(ci * C, C)]],
            rows[ci % NRBUF], gsems[ci % NRBUF])

    gdesc = [None] * NRBUF
    sdesc = [None] * NRBUF
    gdesc[0] = start_gather(0)

    for ci in range(NCHUNK):
        cur = ci % NRBUF
        if ci + 1 < NCHUNK:
            nb = (ci + 1) % NRBUF
            if sdesc[nb] is not None:
                sdesc[nb].wait()
            gdesc[nb] = start_gather(ci + 1)
        gdesc[cur].wait()
        if ci == 0:
            wdesc.wait()

        rbuf = rows[cur]
        h = ci % HALVES
        woff = h * C

        @plsc.parallel_loop(0, C, unroll=2)
        def add_row(r):
            for j in range(VECS_PER_ROW):
                off = j * LANES
                v = wpe_v[woff + r, pl.ds(off, LANES)]
                plsc.addupdate(rbuf.at[r, pl.ds(off, LANES)], v)

        b = ci // HALVES
        dst = b * SEQ + pos_base + woff
        sdesc[cur] = pltpu.async_copy(
            rbuf, out_hbm.at[pl.ds(dst, C)], ssems[cur])

    for k in range(NRBUF - 1):
        sdesc[(NCHUNK - 1 - k) % NRBUF].wait()


@jax.jit
def _emb(flat_ids, wte, wpe):
    mesh = plsc.VectorSubcoreMesh(
        core_axis_name="c", subcore_axis_name="s", num_cores=NC, num_subcores=NS
    )
    return pl.kernel(
        _emb_body,
        out_type=jax.ShapeDtypeStruct((TOT, D), jnp.float32),
        mesh=mesh,
        scratch_types=[
            pltpu.VMEM((BATCH * PPW,), jnp.int32),
            pltpu.VMEM((PPW, D), jnp.float32),
            pltpu.VMEM((C, D), jnp.float32),
            pltpu.VMEM((C, D), jnp.float32),
            pltpu.VMEM((C, D), jnp.float32),
            pltpu.SemaphoreType.DMA,
            pltpu.SemaphoreType.DMA,
            pltpu.SemaphoreType.DMA,
            pltpu.SemaphoreType.DMA,
            pltpu.SemaphoreType.DMA,
            pltpu.SemaphoreType.DMA,
            pltpu.SemaphoreType.DMA,
        ],
    )(flat_ids, wte, wpe)


def kernel(input_ids, wte, wpe):
    flat_ids = input_ids.reshape(TOT)
    out = _emb(flat_ids, wte, wpe)
    return out.reshape(BATCH, SEQ, D)
